# baseline (speedup 1.0000x reference)
import jax
import jax.numpy as jnp
from jax import lax
from jax.experimental import pallas as pl
from jax.experimental.pallas import tpu as pltpu
from jax.experimental.pallas import tpu_sc as plsc

MAX_POSITIONS = 1000000
EMBED_DIM = 32
BATCH = 16384
HIST = 200

N = BATCH * HIST
NW = 32
PER_W = N // NW
IDX_W = 128
CHUNK = 1024
ROWS_PER_CHUNK = CHUNK // IDX_W
CHUNKS_PER_W = PER_W // CHUNK
NBUF = 2


def _sc_gather(idx_hbm, table_hbm, out_hbm, idx_v0, idx_v1, rows_v0,
               rows_v1, isem0, isem1, gsem0, gsem1, wsem0, wsem1):
    idx_v = [idx_v0, idx_v1]
    rows_v = [rows_v0, rows_v1]
    isem = [isem0, isem1]
    gsem = [gsem0, gsem1]
    wsem = [wsem0, wsem1]

    wid = lax.axis_index("s") * 2 + lax.axis_index("c")
    out_base = wid * PER_W

    def start_idx(c, b):
        pltpu.async_copy(
            idx_hbm.at[pl.ds(out_base + c * CHUNK, CHUNK)],
            idx_v[b], isem[b])

    def wait_idx(b):
        pltpu.make_async_copy(
            idx_hbm.at[pl.ds(out_base, CHUNK)],
            idx_v[b], isem[b]).wait()

    def wait_write(b):
        pltpu.make_async_copy(
            rows_v[b], out_hbm.at[pl.ds(out_base, CHUNK)], wsem[b]).wait()

    def step(c, b, first, fetch_next):
        if not first:
            wait_write(b)
        wait_idx(b)
        pltpu.async_copy(table_hbm.at[idx_v[b]], rows_v[b], gsem[b])
        pltpu.make_async_copy(
            table_hbm.at[idx_v[b]], rows_v[b], gsem[b]).wait()
        if fetch_next:
            start_idx(c + NBUF, b)
        pltpu.async_copy(
            rows_v[b], out_hbm.at[pl.ds(out_base + c * CHUNK, CHUNK)],
            wsem[b])

    for b in range(NBUF):
        start_idx(b, b)
    for b in range(NBUF):
        step(b, b, first=True, fetch_next=True)

    def body(g, carry):
        for b in range(NBUF):
            step(g * NBUF + b, b, first=False, fetch_next=True)
        return carry

    lax.fori_loop(1, CHUNKS_PER_W // NBUF - 1, body, 0)

    for b in range(NBUF):
        step(CHUNKS_PER_W - NBUF + b, b, first=False, fetch_next=False)
    for b in range(NBUF):
        wait_write(b)


@jax.jit
def _lookup(idx2d, table):
    mesh = plsc.VectorSubcoreMesh(core_axis_name="c", subcore_axis_name="s")
    f = pl.kernel(
        _sc_gather,
        out_type=jax.ShapeDtypeStruct((N, EMBED_DIM), jnp.float32),
        mesh=mesh,
        scratch_types=[
            pltpu.VMEM((CHUNK,), jnp.int32),
            pltpu.VMEM((CHUNK,), jnp.int32),
            pltpu.VMEM((CHUNK, EMBED_DIM), jnp.float32),
            pltpu.VMEM((CHUNK, EMBED_DIM), jnp.float32),
            pltpu.SemaphoreType.DMA,
            pltpu.SemaphoreType.DMA,
            pltpu.SemaphoreType.DMA,
            pltpu.SemaphoreType.DMA,
            pltpu.SemaphoreType.DMA,
            pltpu.SemaphoreType.DMA,
        ],
        compiler_params=pltpu.CompilerParams(use_tc_tiling_on_sc=False),
    )
    return f(idx2d, table)


def kernel(inputs, embeddings):
    idx2d = inputs.astype(jnp.int32).reshape(N)
    out = _lookup(idx2d, embeddings)
    return out.reshape(BATCH, HIST, EMBED_DIM)


# speedup vs baseline: 1.0037x; 1.0037x over previous
import jax
import jax.numpy as jnp
from jax import lax
from jax.experimental import pallas as pl
from jax.experimental.pallas import tpu as pltpu
from jax.experimental.pallas import tpu_sc as plsc

MAX_POSITIONS = 1000000
EMBED_DIM = 32
BATCH = 16384
HIST = 200

N = BATCH * HIST
NW = 32
PER_W = N // NW
CHUNK = 1024
CHUNKS_PER_W = PER_W // CHUNK
NBUF = 3


def _sc_gather(idx_hbm, table_hbm, out_hbm, idx_v0, idx_v1, idx_v2,
               rows_v0, rows_v1, rows_v2, isem0, isem1, isem2,
               gsem0, gsem1, gsem2, wsem0, wsem1, wsem2):
    idx_v = [idx_v0, idx_v1, idx_v2]
    rows_v = [rows_v0, rows_v1, rows_v2]
    isem = [isem0, isem1, isem2]
    gsem = [gsem0, gsem1, gsem2]
    wsem = [wsem0, wsem1, wsem2]

    wid = lax.axis_index("s") * 2 + lax.axis_index("c")
    out_base = wid * PER_W

    def start_idx(c, b):
        pltpu.async_copy(
            idx_hbm.at[pl.ds(out_base + c * CHUNK, CHUNK)],
            idx_v[b], isem[b])

    def wait_idx(b):
        pltpu.make_async_copy(
            idx_hbm.at[pl.ds(out_base, CHUNK)], idx_v[b], isem[b]).wait()

    def fire_gather(b):
        pltpu.async_copy(table_hbm.at[idx_v[b]], rows_v[b], gsem[b])

    def wait_gather(b):
        pltpu.make_async_copy(
            table_hbm.at[idx_v[b]], rows_v[b], gsem[b]).wait()

    def start_write(c, b):
        pltpu.async_copy(
            rows_v[b], out_hbm.at[pl.ds(out_base + c * CHUNK, CHUNK)],
            wsem[b])

    def wait_write(b):
        pltpu.make_async_copy(
            rows_v[b], out_hbm.at[pl.ds(out_base, CHUNK)], wsem[b]).wait()

    def step(c, b, p, wr_wait, prefetch, prev):
        wait_idx(b)
        if wr_wait:
            wait_write(b)
        fire_gather(b)
        if prev:
            wait_gather(p)
            start_write(c - 1, p)
            if prefetch:
                start_idx(c + NBUF - 1, p)

    for b in range(NBUF):
        start_idx(b, b)
    step(0, 0, 2, wr_wait=False, prefetch=False, prev=False)
    step(1, 1, 0, wr_wait=False, prefetch=True, prev=True)
    step(2, 2, 1, wr_wait=False, prefetch=True, prev=True)

    def body(g, carry):
        c0 = g * NBUF
        step(c0 + 0, 0, 2, wr_wait=True, prefetch=True, prev=True)
        step(c0 + 1, 1, 0, wr_wait=True, prefetch=True, prev=True)
        step(c0 + 2, 2, 1, wr_wait=True, prefetch=True, prev=True)
        return carry

    lax.fori_loop(1, CHUNKS_PER_W // NBUF - 1, body, 0)

    step(96, 0, 2, wr_wait=True, prefetch=True, prev=True)
    step(97, 1, 0, wr_wait=True, prefetch=True, prev=True)
    step(98, 2, 1, wr_wait=True, prefetch=False, prev=True)
    step(99, 0, 2, wr_wait=True, prefetch=False, prev=True)
    wait_gather(0)
    start_write(99, 0)
    for b in range(NBUF):
        wait_write(b)


@jax.jit
def _lookup(idx1d, table):
    mesh = plsc.VectorSubcoreMesh(core_axis_name="c", subcore_axis_name="s")
    f = pl.kernel(
        _sc_gather,
        out_type=jax.ShapeDtypeStruct((N, EMBED_DIM), jnp.float32),
        mesh=mesh,
        scratch_types=(
            [pltpu.VMEM((CHUNK,), jnp.int32) for _ in range(NBUF)]
            + [pltpu.VMEM((CHUNK, EMBED_DIM), jnp.float32)
               for _ in range(NBUF)]
            + [pltpu.SemaphoreType.DMA for _ in range(3 * NBUF)]
        ),
        compiler_params=pltpu.CompilerParams(use_tc_tiling_on_sc=False),
    )
    return f(idx1d, table)


def kernel(inputs, embeddings):
    idx1d = inputs.astype(jnp.int32).reshape(N)
    out = _lookup(idx1d, embeddings)
    return out.reshape(BATCH, HIST, EMBED_DIM)
